# baseline (speedup 1.0000x reference)
import functools

import jax
import jax.numpy as jnp
from jax import lax
from jax.experimental import pallas as pl
from jax.experimental.pallas import tpu as pltpu
from jax.experimental.pallas import tpu_sc as plsc

EMBED_DIM = 32
ROWS_PER_STREAM = 128
K = 8
CHUNK = K * ROWS_PER_STREAM


def _make_gather(total_rows: int):
    info = plsc.get_sparse_core_info()
    nc, ns = info.num_cores, info.num_subcores
    nw = nc * ns
    assert total_rows % (nw * CHUNK) == 0
    rows_per_w = total_rows // nw
    n = rows_per_w // CHUNK

    mesh = plsc.VectorSubcoreMesh(core_axis_name="c", subcore_axis_name="s")

    @functools.partial(
        pl.kernel,
        mesh=mesh,
        out_type=jax.ShapeDtypeStruct((total_rows, EMBED_DIM), jnp.float32),
        scratch_types=[
            pltpu.VMEM((K, ROWS_PER_STREAM), jnp.int32),
            pltpu.VMEM((CHUNK, EMBED_DIM), jnp.float32),
            pltpu.SemaphoreType.DMA,
        ],
        compiler_params=pltpu.CompilerParams(use_tc_tiling_on_sc=False),
    )
    def gather_kernel(table_hbm, idx_hbm, out_hbm, idx_v, rows_v, sem_g):
        wid = lax.axis_index("s") * nc + lax.axis_index("c")
        w_idx_row0 = wid * (rows_per_w // ROWS_PER_STREAM)

        def body(c, carry):
            pltpu.sync_copy(idx_hbm.at[pl.ds(w_idx_row0 + c * K, K)], idx_v)
            for j in range(K):
                pltpu.async_copy(
                    table_hbm.at[idx_v.at[j]],
                    rows_v.at[pl.ds(j * ROWS_PER_STREAM, ROWS_PER_STREAM)],
                    sem_g,
                )
            pltpu.make_async_copy(
                table_hbm.at[pl.ds(0, CHUNK)], rows_v, sem_g
            ).wait()
            base = (w_idx_row0 + c * K) * ROWS_PER_STREAM
            pltpu.sync_copy(rows_v, out_hbm.at[pl.ds(base, CHUNK)])
            return carry

        lax.fori_loop(0, n, body, 0)

    return gather_kernel


def _kernel_impl(field_0, field_1, field_2, table):
    b, l = field_0.shape
    total = 3 * b * l
    idx2d = (
        jnp.stack([field_0, field_1, field_2], axis=-1)
        .astype(jnp.int32)
        .reshape(total // ROWS_PER_STREAM, ROWS_PER_STREAM)
    )
    out_flat = _make_gather(total)(table, idx2d)
    return out_flat.reshape(b, l, 3 * EMBED_DIM)


kernel = jax.jit(_kernel_impl)


# speedup vs baseline: 1.0219x; 1.0219x over previous
import functools

import jax
import jax.numpy as jnp
from jax import lax
from jax.experimental import pallas as pl
from jax.experimental.pallas import tpu as pltpu
from jax.experimental.pallas import tpu_sc as plsc

EMBED_DIM = 32
ROWS_PER_STREAM = 128
K = 8
CHUNK = K * ROWS_PER_STREAM


def _make_gather(total_rows: int):
    info = plsc.get_sparse_core_info()
    nc, ns = info.num_cores, info.num_subcores
    nw = nc * ns
    assert total_rows % (nw * CHUNK) == 0
    rows_per_w = total_rows // nw
    n = rows_per_w // CHUNK
    assert n >= 2

    mesh = plsc.VectorSubcoreMesh(core_axis_name="c", subcore_axis_name="s")

    @functools.partial(
        pl.kernel,
        mesh=mesh,
        out_type=jax.ShapeDtypeStruct((total_rows, EMBED_DIM), jnp.float32),
        scratch_types=[
            pltpu.VMEM((K, ROWS_PER_STREAM), jnp.int32),
            pltpu.VMEM((K, ROWS_PER_STREAM), jnp.int32),
            pltpu.VMEM((CHUNK, EMBED_DIM), jnp.float32),
            pltpu.VMEM((CHUNK, EMBED_DIM), jnp.float32),
            pltpu.SemaphoreType.DMA,
            pltpu.SemaphoreType.DMA,
            pltpu.SemaphoreType.DMA,
            pltpu.SemaphoreType.DMA,
            pltpu.SemaphoreType.DMA,
            pltpu.SemaphoreType.DMA,
        ],
        compiler_params=pltpu.CompilerParams(use_tc_tiling_on_sc=False),
    )
    def gather_kernel(table_hbm, idx_hbm, out_hbm, idx_v0, idx_v1, rows_v0,
                      rows_v1, sem_i0, sem_i1, sem_g0, sem_g1, sem_o0,
                      sem_o1):
        wid = lax.axis_index("s") * nc + lax.axis_index("c")
        w_idx_row0 = wid * (rows_per_w // ROWS_PER_STREAM)
        idx_v = (idx_v0, idx_v1)
        rows_v = (rows_v0, rows_v1)
        sem_i = (sem_i0, sem_i1)
        sem_g = (sem_g0, sem_g1)
        sem_o = (sem_o0, sem_o1)

        def issue_idx(c):
            b = c % 2
            pltpu.async_copy(
                idx_hbm.at[pl.ds(w_idx_row0 + c * K, K)], idx_v[b], sem_i[b]
            )

        def wait_idx(b):
            pltpu.make_async_copy(
                idx_hbm.at[pl.ds(0, K)], idx_v[b], sem_i[b]
            ).wait()

        def fire_gathers(b):
            for j in range(K):
                pltpu.async_copy(
                    table_hbm.at[idx_v[b].at[j]],
                    rows_v[b].at[pl.ds(j * ROWS_PER_STREAM, ROWS_PER_STREAM)],
                    sem_g[b],
                )

        def drain_gathers(b):
            pltpu.make_async_copy(
                table_hbm.at[pl.ds(0, CHUNK)], rows_v[b], sem_g[b]
            ).wait()

        def fire_out(c, b):
            base = (w_idx_row0 + c * K) * ROWS_PER_STREAM
            pltpu.async_copy(
                rows_v[b], out_hbm.at[pl.ds(base, CHUNK)], sem_o[b]
            )

        def wait_out(b):
            pltpu.make_async_copy(
                rows_v[b], out_hbm.at[pl.ds(0, CHUNK)], sem_o[b]
            ).wait()

        pltpu.sync_copy(idx_hbm.at[pl.ds(w_idx_row0, K)], idx_v[0])
        fire_gathers(0)
        issue_idx(1)

        for c in range(n):
            b = c % 2
            if c >= 1:
                wait_out(1 - b)
            if c + 1 < n:
                wait_idx(1 - b)
                fire_gathers(1 - b)
            drain_gathers(b)
            if c + 2 < n:
                issue_idx(c + 2)
            fire_out(c, b)
        wait_out((n - 1) % 2)

    return gather_kernel


def _kernel_impl(field_0, field_1, field_2, table):
    b, l = field_0.shape
    total = 3 * b * l
    idx2d = (
        jnp.stack([field_0, field_1, field_2], axis=-1)
        .astype(jnp.int32)
        .reshape(total // ROWS_PER_STREAM, ROWS_PER_STREAM)
    )
    out_flat = _make_gather(total)(table, idx2d)
    return out_flat.reshape(b, l, 3 * EMBED_DIM)


kernel = jax.jit(_kernel_impl)


# speedup vs baseline: 1.0230x; 1.0010x over previous
import functools

import jax
import jax.numpy as jnp
from jax import lax
from jax.experimental import layout as jax_layout
from jax.experimental import pallas as pl
from jax.experimental.pallas import tpu as pltpu
from jax.experimental.pallas import tpu_sc as plsc

EMBED_DIM = 32
ROWS_PER_STREAM = 128
K = 8
CHUNK = K * ROWS_PER_STREAM


def _make_gather(total_rows: int):
    info = plsc.get_sparse_core_info()
    nc, ns = info.num_cores, info.num_subcores
    nw = nc * ns
    assert total_rows % (nw * CHUNK) == 0
    rows_per_w = total_rows // nw
    n = rows_per_w // CHUNK
    assert n >= 2

    mesh = plsc.VectorSubcoreMesh(core_axis_name="c", subcore_axis_name="s")

    @functools.partial(
        pl.kernel,
        mesh=mesh,
        out_type=jax.ShapeDtypeStruct((total_rows, EMBED_DIM), jnp.float32),
        scratch_types=[
            pltpu.VMEM((K, ROWS_PER_STREAM), jnp.int32),
            pltpu.VMEM((K, ROWS_PER_STREAM), jnp.int32),
            pltpu.VMEM((CHUNK, EMBED_DIM), jnp.float32),
            pltpu.VMEM((CHUNK, EMBED_DIM), jnp.float32),
            pltpu.SemaphoreType.DMA,
            pltpu.SemaphoreType.DMA,
            pltpu.SemaphoreType.DMA,
            pltpu.SemaphoreType.DMA,
            pltpu.SemaphoreType.DMA,
            pltpu.SemaphoreType.DMA,
        ],
        compiler_params=pltpu.CompilerParams(use_tc_tiling_on_sc=False),
    )
    def gather_kernel(table_hbm, idx_hbm, out_hbm, idx_v0, idx_v1, rows_v0,
                      rows_v1, sem_i0, sem_i1, sem_g0, sem_g1, sem_o0,
                      sem_o1):
        wid = lax.axis_index("s") * nc + lax.axis_index("c")
        w_idx_row0 = wid * (rows_per_w // ROWS_PER_STREAM)
        idx_v = (idx_v0, idx_v1)
        rows_v = (rows_v0, rows_v1)
        sem_i = (sem_i0, sem_i1)
        sem_g = (sem_g0, sem_g1)
        sem_o = (sem_o0, sem_o1)

        def issue_idx(c):
            b = c % 2
            pltpu.async_copy(
                idx_hbm.at[pl.ds(w_idx_row0 + c * K, K)], idx_v[b], sem_i[b]
            )

        def wait_idx(b):
            pltpu.make_async_copy(
                idx_hbm.at[pl.ds(0, K)], idx_v[b], sem_i[b]
            ).wait()

        def fire_gathers(b):
            for j in range(K):
                pltpu.async_copy(
                    table_hbm.at[idx_v[b].at[j]],
                    rows_v[b].at[pl.ds(j * ROWS_PER_STREAM, ROWS_PER_STREAM)],
                    sem_g[b],
                )

        def drain_gathers(b):
            pltpu.make_async_copy(
                table_hbm.at[pl.ds(0, CHUNK)], rows_v[b], sem_g[b]
            ).wait()

        def fire_out(c, b):
            base = (w_idx_row0 + c * K) * ROWS_PER_STREAM
            pltpu.async_copy(
                rows_v[b], out_hbm.at[pl.ds(base, CHUNK)], sem_o[b]
            )

        def wait_out(b):
            pltpu.make_async_copy(
                rows_v[b], out_hbm.at[pl.ds(0, CHUNK)], sem_o[b]
            ).wait()

        pltpu.sync_copy(idx_hbm.at[pl.ds(w_idx_row0, K)], idx_v[0])
        fire_gathers(0)
        issue_idx(1)

        for c in range(n):
            b = c % 2
            if c >= 1:
                wait_out(1 - b)
            if c + 1 < n:
                wait_idx(1 - b)
                fire_gathers(1 - b)
            drain_gathers(b)
            if c + 2 < n:
                issue_idx(c + 2)
            fire_out(c, b)
        wait_out((n - 1) % 2)

    return gather_kernel


def _kernel_impl(field_0, field_1, field_2, table):
    b, l = field_0.shape
    total = 3 * b * l
    idx2d = (
        jnp.stack([field_0, field_1, field_2], axis=-1)
        .astype(jnp.int32)
        .reshape(total // ROWS_PER_STREAM, ROWS_PER_STREAM)
    )
    out_flat = _make_gather(total)(table, idx2d)
    return out_flat.reshape(b, l, 3 * EMBED_DIM)


@functools.lru_cache(maxsize=None)
def _jitted():
    fmt = jax_layout.Format(
        jax_layout.Layout(major_to_minor=(0, 1, 2), tiling=()),
        jax.sharding.SingleDeviceSharding(jax.devices()[0]),
    )
    return jax.jit(_kernel_impl, out_shardings=fmt)


def kernel(field_0, field_1, field_2, table):
    return _jitted()(field_0, field_1, field_2, table)


# speedup vs baseline: 2.4249x; 2.3704x over previous
import functools

import jax
import jax.numpy as jnp
from jax import lax
from jax.experimental import pallas as pl
from jax.experimental.pallas import tpu as pltpu
from jax.experimental.pallas import tpu_sc as plsc

EMBED_DIM = 32
NF = 3
ROWS_PER_STREAM = 128
Q = 4
CHUNK_POS = Q * ROWS_PER_STREAM


def _make_gather(n_pos: int):
    info = plsc.get_sparse_core_info()
    nc, ns = info.num_cores, info.num_subcores
    nw = nc * ns
    assert n_pos % (nw * CHUNK_POS) == 0
    pos_per_w = n_pos // nw
    n = pos_per_w // CHUNK_POS
    idx_rows_per_w = pos_per_w // ROWS_PER_STREAM

    mesh = plsc.VectorSubcoreMesh(core_axis_name="c", subcore_axis_name="s")

    @functools.partial(
        pl.kernel,
        mesh=mesh,
        out_type=jax.ShapeDtypeStruct((n_pos, 4 * EMBED_DIM), jnp.float32),
        scratch_types=[
            pltpu.VMEM((NF, Q, ROWS_PER_STREAM), jnp.int32),
            pltpu.VMEM((NF, CHUNK_POS, EMBED_DIM), jnp.float32),
            pltpu.SemaphoreType.DMA,
            pltpu.SemaphoreType.DMA,
            pltpu.SemaphoreType.DMA,
        ],
        compiler_params=pltpu.CompilerParams(use_tc_tiling_on_sc=False),
    )
    def gather_kernel(table_hbm, idx_hbm, out_hbm, idx_v, stg, sem0, sem1,
                      sem2):
        wid = lax.axis_index("s") * nc + lax.axis_index("c")
        w_row0 = wid * idx_rows_per_w
        w_pos0 = wid * pos_per_w
        sems = (sem0, sem1, sem2)

        def body(c, carry):
            for f in range(NF):
                pltpu.sync_copy(
                    idx_hbm.at[f, pl.ds(w_row0 + c * Q, Q)], idx_v.at[f]
                )
            for f in range(NF):
                for q in range(Q):
                    pltpu.async_copy(
                        table_hbm.at[idx_v.at[f, q]],
                        stg.at[
                            f,
                            pl.ds(q * ROWS_PER_STREAM, ROWS_PER_STREAM),
                        ],
                        sems[f],
                    )
            for f in range(NF):
                pltpu.make_async_copy(
                    table_hbm.at[pl.ds(0, CHUNK_POS)],
                    stg.at[f],
                    sems[f],
                ).wait()
            base = w_pos0 + c * CHUNK_POS
            for f in range(NF):
                pltpu.sync_copy(
                    stg.at[f],
                    out_hbm.at[
                        pl.ds(base, CHUNK_POS),
                        pl.ds(f * EMBED_DIM, EMBED_DIM),
                    ],
                )
            return carry

        lax.fori_loop(0, n, body, 0)

    return gather_kernel


def _lane_slice_kernel(x_ref, o_ref):
    o_ref[...] = x_ref[...].reshape(8, 200, 128)[:, :, : NF * EMBED_DIM]


def _kernel_impl(field_0, field_1, field_2, table):
    b, l = field_0.shape
    n_pos = b * l
    idx3 = (
        jnp.stack([field_0, field_1, field_2])
        .astype(jnp.int32)
        .reshape(NF, n_pos // ROWS_PER_STREAM, ROWS_PER_STREAM)
    )
    padded = _make_gather(n_pos)(table, idx3)
    return pl.pallas_call(
        _lane_slice_kernel,
        grid=(b // 8,),
        in_specs=[pl.BlockSpec((8 * l, 128), lambda i: (i, 0))],
        out_specs=pl.BlockSpec((8, l, NF * EMBED_DIM), lambda i: (i, 0, 0)),
        out_shape=jax.ShapeDtypeStruct((b, l, NF * EMBED_DIM), jnp.float32),
    )(padded)


kernel = jax.jit(_kernel_impl)


# speedup vs baseline: 2.7106x; 1.1178x over previous
import functools

import jax
import jax.numpy as jnp
from jax import lax
from jax.experimental import pallas as pl
from jax.experimental.pallas import tpu as pltpu
from jax.experimental.pallas import tpu_sc as plsc

EMBED_DIM = 32
NF = 3
ROWS_PER_STREAM = 128
Q = 4
CHUNK_POS = Q * ROWS_PER_STREAM


def _make_gather(n_pos: int):
    info = plsc.get_sparse_core_info()
    nc, ns = info.num_cores, info.num_subcores
    nw = nc * ns
    assert n_pos % (nw * CHUNK_POS) == 0
    pos_per_w = n_pos // nw
    n = pos_per_w // CHUNK_POS
    idx_rows_per_w = pos_per_w // ROWS_PER_STREAM

    mesh = plsc.VectorSubcoreMesh(core_axis_name="c", subcore_axis_name="s")

    @functools.partial(
        pl.kernel,
        mesh=mesh,
        out_type=jax.ShapeDtypeStruct((n_pos, 4 * EMBED_DIM), jnp.float32),
        scratch_types=[
            pltpu.VMEM((NF, Q, ROWS_PER_STREAM), jnp.int32),
            pltpu.VMEM((NF, CHUNK_POS, EMBED_DIM), jnp.float32),
            pltpu.SemaphoreType.DMA,
            pltpu.SemaphoreType.DMA,
            pltpu.SemaphoreType.DMA,
        ],
        compiler_params=pltpu.CompilerParams(use_tc_tiling_on_sc=False),
    )
    def gather_kernel(table_hbm, idx_hbm, out_hbm, idx_v, stg, sem0, sem1,
                      sem2):
        wid = lax.axis_index("s") * nc + lax.axis_index("c")
        w_row0 = wid * idx_rows_per_w
        w_pos0 = wid * pos_per_w
        sems = (sem0, sem1, sem2)

        def body(c, carry):
            for f in range(NF):
                pltpu.sync_copy(
                    idx_hbm.at[f, pl.ds(w_row0 + c * Q, Q)], idx_v.at[f]
                )
            for f in range(NF):
                for q in range(Q):
                    pltpu.async_copy(
                        table_hbm.at[idx_v.at[f, q]],
                        stg.at[
                            f,
                            pl.ds(q * ROWS_PER_STREAM, ROWS_PER_STREAM),
                        ],
                        sems[f],
                    )
            for f in range(NF):
                pltpu.make_async_copy(
                    table_hbm.at[pl.ds(0, CHUNK_POS)],
                    stg.at[f],
                    sems[f],
                ).wait()
            base = w_pos0 + c * CHUNK_POS
            for f in range(NF):
                pltpu.sync_copy(
                    stg.at[f],
                    out_hbm.at[
                        pl.ds(base, CHUNK_POS),
                        pl.ds(f * EMBED_DIM, EMBED_DIM),
                    ],
                )
            return carry

        lax.fori_loop(0, n, body, 0)

    return gather_kernel


def _lane_slice_kernel(x_ref, o_ref):
    o_ref[...] = x_ref[...].reshape(32, 200, 128)[:, :, : NF * EMBED_DIM]


def _kernel_impl(field_0, field_1, field_2, table):
    b, l = field_0.shape
    n_pos = b * l
    idx3 = (
        jnp.stack([field_0, field_1, field_2])
        .astype(jnp.int32)
        .reshape(NF, n_pos // ROWS_PER_STREAM, ROWS_PER_STREAM)
    )
    padded = _make_gather(n_pos)(table, idx3)
    return pl.pallas_call(
        _lane_slice_kernel,
        grid=(b // 32,),
        in_specs=[pl.BlockSpec((32 * l, 128), lambda i: (i, 0))],
        out_specs=pl.BlockSpec((32, l, NF * EMBED_DIM), lambda i: (i, 0, 0)),
        out_shape=jax.ShapeDtypeStruct((b, l, NF * EMBED_DIM), jnp.float32),
    )(padded)


kernel = jax.jit(_kernel_impl)


# speedup vs baseline: 2.7140x; 1.0013x over previous
import functools

import jax
import jax.numpy as jnp
from jax import lax
from jax.experimental import pallas as pl
from jax.experimental.pallas import tpu as pltpu
from jax.experimental.pallas import tpu_sc as plsc

EMBED_DIM = 32
NF = 3
ROWS_PER_STREAM = 128
Q = 4
CHUNK_POS = Q * ROWS_PER_STREAM


def _make_gather(n_pos: int):
    info = plsc.get_sparse_core_info()
    nc, ns = info.num_cores, info.num_subcores
    nw = nc * ns
    assert n_pos % (nw * CHUNK_POS) == 0
    pos_per_w = n_pos // nw
    n = pos_per_w // CHUNK_POS
    idx_rows_per_w = pos_per_w // ROWS_PER_STREAM

    mesh = plsc.VectorSubcoreMesh(core_axis_name="c", subcore_axis_name="s")

    @functools.partial(
        pl.kernel,
        mesh=mesh,
        out_type=jax.ShapeDtypeStruct((n_pos, 4 * EMBED_DIM), jnp.float32),
        scratch_types=[
            pltpu.VMEM((NF, Q, ROWS_PER_STREAM), jnp.int32),
            pltpu.VMEM((NF, CHUNK_POS, EMBED_DIM), jnp.float32),
            pltpu.SemaphoreType.DMA,
            pltpu.SemaphoreType.DMA,
            pltpu.SemaphoreType.DMA,
        ],
        compiler_params=pltpu.CompilerParams(use_tc_tiling_on_sc=False),
    )
    def gather_kernel(table_hbm, idx_hbm, out_hbm, idx_v, stg, sem0, sem1,
                      sem2):
        wid = lax.axis_index("s") * nc + lax.axis_index("c")
        w_row0 = wid * idx_rows_per_w
        w_pos0 = wid * pos_per_w
        sems = (sem0, sem1, sem2)

        def body(c, carry):
            for f in range(NF):
                pltpu.sync_copy(
                    idx_hbm.at[f, pl.ds(w_row0 + c * Q, Q)], idx_v.at[f]
                )
            for f in range(NF):
                for q in range(Q):
                    pltpu.async_copy(
                        table_hbm.at[idx_v.at[f, q]],
                        stg.at[
                            f,
                            pl.ds(q * ROWS_PER_STREAM, ROWS_PER_STREAM),
                        ],
                        sems[f],
                    )
            for f in range(NF):
                pltpu.make_async_copy(
                    table_hbm.at[pl.ds(0, CHUNK_POS)],
                    stg.at[f],
                    sems[f],
                ).wait()
            base = w_pos0 + c * CHUNK_POS
            for f in range(NF):
                pltpu.sync_copy(
                    stg.at[f],
                    out_hbm.at[
                        pl.ds(base, CHUNK_POS),
                        pl.ds(f * EMBED_DIM, EMBED_DIM),
                    ],
                )
            return carry

        lax.fori_loop(0, n, body, 0)

    return gather_kernel


def _lane_slice_kernel(x_ref, o_ref):
    o_ref[...] = x_ref[...].reshape(64, 200, 128)[:, :, : NF * EMBED_DIM]


def _kernel_impl(field_0, field_1, field_2, table):
    b, l = field_0.shape
    n_pos = b * l
    idx3 = (
        jnp.stack([field_0, field_1, field_2])
        .astype(jnp.int32)
        .reshape(NF, n_pos // ROWS_PER_STREAM, ROWS_PER_STREAM)
    )
    padded = _make_gather(n_pos)(table, idx3)
    return pl.pallas_call(
        _lane_slice_kernel,
        grid=(b // 64,),
        in_specs=[pl.BlockSpec((64 * l, 128), lambda i: (i, 0))],
        out_specs=pl.BlockSpec((64, l, NF * EMBED_DIM), lambda i: (i, 0, 0)),
        out_shape=jax.ShapeDtypeStruct((b, l, NF * EMBED_DIM), jnp.float32),
    )(padded)


kernel = jax.jit(_kernel_impl)
